# baseline (speedup 1.0000x reference)
import jax
import jax.numpy as jnp
from jax import lax
from jax.experimental import pallas as pl
from jax.experimental.pallas import tpu as pltpu
from jax.experimental.pallas import tpu_sc as plsc

N_Q = 8
HID = 1024
T = 2048
GROUPS = HID // 16

TC_SLABS = 16
TC_ROWS = TC_SLABS * T
C = 2048
NB = 2
CH_TC = TC_ROWS // C


def _tc_kernel(x_hbm, emb_hbm, o_hbm, emb_v, *bufs_and_sems):
    ibufs = bufs_and_sems[0:NB]
    obufs = bufs_and_sems[NB:2 * NB]
    sem_e = bufs_and_sems[2 * NB]
    sins = bufs_and_sems[2 * NB + 1:2 * NB + 1 + NB]
    souts = bufs_and_sems[2 * NB + 1 + NB:]

    def in_copy(b, i):
        return pltpu.make_async_copy(
            x_hbm.at[pl.ds(i * C, C), :], ibufs[b], sins[b])

    def out_copy(b, i):
        return pltpu.make_async_copy(
            obufs[b], o_hbm.at[pl.ds(i * C, C), :], souts[b])

    pltpu.make_async_copy(emb_hbm, emb_v, sem_e).start()
    for b in range(NB):
        in_copy(b, b).start()
    pltpu.make_async_copy(emb_hbm, emb_v, sem_e).wait()

    def outer(k, _):
        for b in range(NB):
            i = k * NB + b
            in_copy(b, i).wait()

            @pl.when(i >= NB)
            def _():
                out_copy(b, i - NB).wait()

            q = (i * C // T) % N_Q
            obufs[b][...] = ibufs[b][...] + emb_v[pl.ds(q, 1), :]
            out_copy(b, i).start()

            @pl.when(i + NB < CH_TC)
            def _():
                in_copy(b, i + NB).start()

        return 0

    jax.lax.fori_loop(0, CH_TC // NB, outer, 0)

    for b in range(NB):
        out_copy(b, CH_TC - NB + b).wait()


def _tc_call(xf_part, emb_table):
    return pl.pallas_call(
        _tc_kernel,
        in_specs=[
            pl.BlockSpec(memory_space=pl.ANY),
            pl.BlockSpec(memory_space=pl.ANY),
        ],
        out_specs=pl.BlockSpec(memory_space=pl.ANY),
        out_shape=jax.ShapeDtypeStruct((TC_ROWS, HID), xf_part.dtype),
        scratch_shapes=(
            [pltpu.VMEM((N_Q, HID), jnp.float32)]
            + [pltpu.VMEM((C, HID), jnp.float32) for _ in range(2 * NB)]
            + [pltpu.SemaphoreType.DMA for _ in range(2 * NB + 1)]
        ),
    )(xf_part, emb_table)


ROWS_PER_W = 1024
R = 16
NIN = 2
NOUT = 4
CH = ROWS_PER_W // R


def _sc_kernel(x_hbm, emb_hbm, out_hbm, emb_v, ib0, ib1, ob0, ob1, ob2, ob3,
               sin0, sin1, sout0, sout1, sout2, sout3):
    ibufs = (ib0, ib1)
    obufs = (ob0, ob1, ob2, ob3)
    sins = (sin0, sin1)
    souts = (sout0, sout1, sout2, sout3)

    c = lax.axis_index("c")
    s = lax.axis_index("s")
    wid = s * 2 + c
    base = wid * ROWS_PER_W
    q = (wid // 2) % N_Q

    pltpu.sync_copy(emb_hbm.at[q], emb_v)

    def start_in(b, i):
        pltpu.async_copy(x_hbm.at[pl.ds(TC_ROWS + base + i * R, R)],
                         ibufs[b], sins[b])

    def wait_in(b):
        pltpu.make_async_copy(x_hbm.at[pl.ds(0, R)], ibufs[b], sins[b]).wait()

    def start_out(b, i):
        pltpu.async_copy(obufs[b], out_hbm.at[pl.ds(base + i * R, R)],
                         souts[b])

    def wait_out(b):
        pltpu.make_async_copy(obufs[b], out_hbm.at[pl.ds(0, R)],
                              souts[b]).wait()

    def compute(bi, bo):
        ib = ibufs[bi]
        ob = obufs[bo]
        for half in range(2):
            g0 = half * (GROUPS // 2)
            embv = [emb_v[pl.ds((g0 + g) * 16, 16)]
                    for g in range(GROUPS // 2)]

            def row_body(r, _):
                for g in range(GROUPS // 2):
                    sl = pl.ds((g0 + g) * 16, 16)
                    ob[r, sl] = ib[r, sl] + embv[g]
                return 0

            lax.fori_loop(0, R, row_body, 0)

    for b in range(NIN):
        start_in(b, b)

    def outer(k, _):
        for j in range(NOUT):
            i = k * NOUT + j
            bi = j % NIN
            bo = j
            wait_in(bi)

            @pl.when(i >= NOUT)
            def _():
                wait_out(bo)

            compute(bi, bo)
            start_out(bo, i)

            @pl.when(i + NIN < CH)
            def _():
                start_in(bi, i + NIN)

        return 0

    lax.fori_loop(0, CH // NOUT, outer, 0)

    for b in range(NOUT):
        wait_out(b)


def _sc_call(xf_part, emb_table):
    mesh = plsc.VectorSubcoreMesh(core_axis_name="c", subcore_axis_name="s")
    run = pl.kernel(
        _sc_kernel,
        mesh=mesh,
        out_type=jax.ShapeDtypeStruct((32 * ROWS_PER_W, HID), xf_part.dtype),
        scratch_types=(
            [pltpu.VMEM((HID,), jnp.float32)]
            + [pltpu.VMEM((R, HID), jnp.float32) for _ in range(NIN + NOUT)]
            + [pltpu.SemaphoreType.DMA for _ in range(NIN + NOUT)]
        ),
    )
    return run(xf_part, emb_table)


def kernel(x, emb_table):
    b, q, t, h = x.shape
    xf = x.reshape(b * q * t, h)
    out_tc = _tc_call(xf, emb_table)
    out_sc = _sc_call(xf, emb_table)
    out = jnp.concatenate([out_tc, out_sc], axis=0)
    return out.reshape(b, q, t, h)


# speedup vs baseline: 2.1704x; 2.1704x over previous
import jax
import jax.numpy as jnp
from jax.experimental import pallas as pl
from jax.experimental.pallas import tpu as pltpu

N_Q = 8
HID = 1024
T = 2048
C = 2048
NB = 2
ROWS = 32 * T
CH = ROWS // C
PER_SLAB = T // C


def _add_kernel(x_hbm, emb_hbm, o_hbm, emb_v, *bufs_and_sems):
    ibufs = bufs_and_sems[0:NB]
    obufs = bufs_and_sems[NB:2 * NB]
    sem_e = bufs_and_sems[2 * NB]
    sins = bufs_and_sems[2 * NB + 1:2 * NB + 1 + NB]
    souts = bufs_and_sems[2 * NB + 1 + NB:]

    def in_copy(b, i):
        return pltpu.make_async_copy(
            x_hbm.at[pl.ds(i * C, C), :], ibufs[b], sins[b])

    def out_copy(b, i):
        return pltpu.make_async_copy(
            obufs[b], o_hbm.at[pl.ds(i * C, C), :], souts[b])

    pltpu.make_async_copy(emb_hbm, emb_v, sem_e).start()
    for b in range(NB):
        in_copy(b, b).start()
    pltpu.make_async_copy(emb_hbm, emb_v, sem_e).wait()

    def outer(k, _):
        for b in range(NB):
            i = k * NB + b
            in_copy(b, i).wait()

            @pl.when(i >= NB)
            def _():
                out_copy(b, i - NB).wait()

            q = (i // PER_SLAB) % N_Q
            obufs[b][...] = ibufs[b][...] + emb_v[pl.ds(q, 1), :]
            out_copy(b, i).start()

            @pl.when(i + NB < CH)
            def _():
                in_copy(b, i + NB).start()

        return 0

    jax.lax.fori_loop(0, CH // NB, outer, 0)

    for b in range(NB):
        out_copy(b, CH - NB + b).wait()


def kernel(x, emb_table):
    b, q, t, h = x.shape
    xf = x.reshape(b * q * t, h)
    out = pl.pallas_call(
        _add_kernel,
        in_specs=[
            pl.BlockSpec(memory_space=pl.ANY),
            pl.BlockSpec(memory_space=pl.ANY),
        ],
        out_specs=pl.BlockSpec(memory_space=pl.ANY),
        out_shape=jax.ShapeDtypeStruct((b * q * t, h), x.dtype),
        scratch_shapes=(
            [pltpu.VMEM((N_Q, h), jnp.float32)]
            + [pltpu.VMEM((C, h), jnp.float32) for _ in range(2 * NB)]
            + [pltpu.SemaphoreType.DMA for _ in range(2 * NB + 1)]
        ),
    )(xf, emb_table)
    return out.reshape(b, q, t, h)
